# baseline (speedup 1.0000x reference)
import functools

import jax
import jax.numpy as jnp
from jax import lax
from jax.experimental import pallas as pl
from jax.experimental.pallas import tpu as pltpu
from jax.experimental.pallas import tpu_sc as plsc

_NC = 2
_NS = 16
_NW = _NC * _NS
_LANES = 16

_CHUNK = 32

_MESH = dict(core_axis_name="c", subcore_axis_name="s")
_PARAMS = pltpu.CompilerParams(needs_layout_passes=False,
                               disable_bounds_checks=True)


def _relayout_kernel(V, D):
  ngf = -(-V // 128)
  vpad = ngf * 128
  base_n = ngf // _NW
  extra = ngf - base_n * _NW

  @functools.partial(
      pl.kernel,
      out_type=jax.ShapeDtypeStruct((vpad // 2, 2 * D), jnp.float32),
      mesh=plsc.VectorSubcoreMesh(**_MESH),
      compiler_params=_PARAMS,
      scratch_types=[
          pltpu.VMEM((2, D, 128), jnp.float32),
          pltpu.VMEM((D, 129), jnp.float32),
          pltpu.VMEM((2, 64, 2 * D), jnp.float32),
          pltpu.SemaphoreType.DMA,
          pltpu.SemaphoreType.DMA,
      ],
  )
  def k(tab_hbm, scr_hbm, slab_v, work_v, outb_v, sem_in, sem_out):
    wid = lax.axis_index("s") * _NC + lax.axis_index("c")
    nslab = base_n + jnp.where(wid < extra, 1, 0)
    rows_k = [lax.iota(jnp.int32, _LANES) + k * _LANES for k in range(D // _LANES)]

    def fire_in(i):
      vg = wid + i * _NW
      p = i & 1
      pltpu.async_copy(
          tab_hbm.at[pl.ds(0, D), pl.ds(vg * 128, 128)],
          slab_v.at[p], sem_in)

    def wait_in(p):
      pltpu.make_async_copy(
          tab_hbm.at[pl.ds(0, D), pl.ds(0, 128)], slab_v.at[p], sem_in).wait()

    def wait_out():
      pltpu.make_async_copy(
          outb_v.at[0], scr_hbm.at[pl.ds(0, 64), :], sem_out).wait()

    def transpose(p, nrows):
      slab_p = slab_v.at[p]
      outb_p = outb_v.at[p]
      for j in range(nrows):
        for h in range(2):
          col = jnp.full((_LANES,), 2 * j + h, jnp.int32)
          for kk in range(D // _LANES):
            x = plsc.load_gather(slab_p, [rows_k[kk], col])
            outb_p[j, pl.ds(h * D + kk * _LANES, _LANES)] = x

    fire_in(0)

    def loop_body(i, _):
      p = i & 1

      @pl.when(i + 1 < nslab)
      def _():
        fire_in(i + 1)

      wait_in(p)

      @pl.when(i >= 2)
      def _():
        wait_out()

      transpose(p, 64)
      vg = wid + i * _NW
      pltpu.async_copy(outb_v.at[p], scr_hbm.at[pl.ds(vg * 64, 64), :], sem_out)
      return ()

    lax.fori_loop(0, nslab, loop_body, ())

    @pl.when(nslab >= 2)
    def _():
      wait_out()

    @pl.when(nslab >= 1)
    def _():
      wait_out()

  return k


def _gather_dot_kernel(B, L, D):
  nb_per_w = B // _NW
  n_chunks = nb_per_w // _CHUNK
  rows_per_chunk = _CHUNK * L
  n_idx_rows = rows_per_chunk // 128
  nk = D // _LANES

  @functools.partial(
      pl.kernel,
      out_type=jax.ShapeDtypeStruct((B * L,), jnp.float32),
      mesh=plsc.VectorSubcoreMesh(**_MESH),
      compiler_params=_PARAMS,
      scratch_types=[
          pltpu.VMEM((_CHUNK + _LANES,), jnp.int32),
          pltpu.VMEM((_CHUNK,), jnp.int32),
          pltpu.VMEM((rows_per_chunk + _LANES,), jnp.int32),
          pltpu.VMEM((n_idx_rows, 128), jnp.int32),
          pltpu.VMEM((_CHUNK, 2 * D), jnp.float32),
          pltpu.VMEM((rows_per_chunk, 2 * D), jnp.float32),
          pltpu.VMEM((rows_per_chunk,), jnp.float32),
          pltpu.SemaphoreType.DMA,
          pltpu.SemaphoreType.DMA,
      ],
  )
  def k(center_hbm, con_hbm, scru_hbm, scrv_hbm, out_hbm,
        craw_v, cidx_v, uraw_v, uidx_v, vrows_v, urows_v, res_v,
        sem_u, sem_v):
    wid = lax.axis_index("s") * _NC + lax.axis_index("c")
    lane = lax.iota(jnp.int32, _LANES)
    lane15 = lane == (_LANES - 1)
    perms = [(lane ^ (1 << p)).reshape(_LANES, 1) for p in range(4)]
    gdims = lax.GatherDimensionNumbers(
        offset_dims=(), collapsed_slice_dims=(0,), start_index_map=(0,))

    def lane_sum(x):
      for p in perms:
        x = x + lax.gather(x, p, gdims, (1,),
                           mode=lax.GatherScatterMode.PROMISE_IN_BOUNDS)
      return x

    def chunk_body(c, _):
      b_base = wid * nb_per_w + c * _CHUNK
      pltpu.sync_copy(center_hbm.at[pl.ds(b_base, _CHUNK)],
                      craw_v.at[pl.ds(0, _CHUNK)])
      pltpu.sync_copy(con_hbm.at[pl.ds(b_base * L, rows_per_chunk)],
                      uraw_v.at[pl.ds(0, rows_per_chunk)])
      for t in range(_CHUNK // _LANES):
        s = pl.ds(t * _LANES, _LANES)
        cidx_v[s] = craw_v[s] >> 1
      for r in range(n_idx_rows):
        for t in range(128 // _LANES):
          uidx_v[r, pl.ds(t * _LANES, _LANES)] = (
              uraw_v[pl.ds(r * 128 + t * _LANES, _LANES)] >> 1)
      cp_v = pltpu.async_copy(scrv_hbm.at[cidx_v], vrows_v, sem_v)
      for j in range(n_idx_rows):
        pltpu.async_copy(scru_hbm.at[uidx_v.at[j]],
                         urows_v.at[pl.ds(j * 128, 128)], sem_u)
      cp_v.wait()
      pltpu.make_async_copy(
          scru_hbm.at[pl.ds(0, rows_per_chunk), :], urows_v, sem_u).wait()

      def dot_body(b, _):
        vbase = (craw_v[pl.ds(b, _LANES)][0] & 1) * D
        vr = [vrows_v[b, pl.ds(vbase + kk * _LANES, _LANES)] for kk in range(nk)]
        upar0 = (uraw_v[pl.ds(b * L, _LANES)] & 1) * D
        upar1 = (uraw_v[pl.ds(b * L + _LANES, _LANES)] & 1) * D
        for l in range(L):
          row = b * L + l
          ubase = upar0[l] if l < _LANES else upar1[l - _LANES]
          acc = urows_v[row, pl.ds(ubase, _LANES)] * vr[0]
          for kk in range(1, nk):
            acc += urows_v[row, pl.ds(ubase + kk * _LANES, _LANES)] * vr[kk]
          tot = lane_sum(acc)
          plsc.store_scatter(
              res_v, [jnp.full((_LANES,), row, jnp.int32)], tot, mask=lane15)
        return ()

      lax.fori_loop(0, _CHUNK, dot_body, ())
      pltpu.sync_copy(res_v, out_hbm.at[pl.ds(b_base * L, rows_per_chunk)])
      return ()

    lax.fori_loop(0, n_chunks, chunk_body, ())

  return k


def kernel(center, con_neg, emb_u, emb_v):
  B, L = con_neg.shape
  V, D = emb_u.shape
  assert B % (_NW * _CHUNK) == 0 and (_CHUNK * L) % 128 == 0
  assert D % _LANES == 0 and V % 2 == 0
  center_flat = center.reshape(B).astype(jnp.int32)
  con_flat = con_neg.reshape(B * L).astype(jnp.int32)
  relayout = _relayout_kernel(V, D)
  scr_u = relayout(emb_u.T)
  scr_v = relayout(emb_v.T)
  out = _gather_dot_kernel(B, L, D)(center_flat, con_flat, scr_u, scr_v)
  return out.reshape(B, 1, L)


# speedup vs baseline: 1.0046x; 1.0046x over previous
import functools

import jax
import jax.numpy as jnp
from jax import lax
from jax.experimental import pallas as pl
from jax.experimental.pallas import tpu as pltpu
from jax.experimental.pallas import tpu_sc as plsc

_NC = 2
_NS = 16
_NW = _NC * _NS
_LANES = 16

_CHUNK = 32

_MESH = dict(core_axis_name="c", subcore_axis_name="s")
_PARAMS = pltpu.CompilerParams(needs_layout_passes=False,
                               disable_bounds_checks=True)


def _relayout_kernel(V, D):
  ngf = -(-V // 128)
  vpad = ngf * 128
  base_n = ngf // _NW
  extra = ngf - base_n * _NW

  @functools.partial(
      pl.kernel,
      out_type=jax.ShapeDtypeStruct((vpad // 2, 2 * D), jnp.float32),
      mesh=plsc.VectorSubcoreMesh(**_MESH),
      compiler_params=_PARAMS,
      scratch_types=[
          pltpu.VMEM((8, D, 128), jnp.float32),
          pltpu.VMEM((4, 64, 2 * D), jnp.float32),
          pltpu.SemaphoreType.DMA,
          pltpu.SemaphoreType.DMA,
      ],
  )
  def k(tab_hbm, scr_hbm, slab_v, outb_v, sem_in, sem_out):
    wid = lax.axis_index("s") * _NC + lax.axis_index("c")
    nslab = base_n + jnp.where(wid < extra, 1, 0)
    rows_k = [lax.iota(jnp.int32, _LANES) + k * _LANES for k in range(D // _LANES)]

    def fire_in(i):
      vg = wid + i * _NW
      pltpu.async_copy(
          tab_hbm.at[pl.ds(0, D), pl.ds(vg * 128, 128)],
          slab_v.at[i & 7], sem_in)

    def wait_in(p):
      pltpu.make_async_copy(
          tab_hbm.at[pl.ds(0, D), pl.ds(0, 128)], slab_v.at[p], sem_in).wait()

    def wait_out():
      pltpu.make_async_copy(
          outb_v.at[0], scr_hbm.at[pl.ds(0, 64), :], sem_out).wait()

    def transpose(p, q, nrows):
      slab_p = slab_v.at[p]
      outb_p = outb_v.at[q]
      for j in range(nrows):
        for h in range(2):
          col = jnp.full((_LANES,), 2 * j + h, jnp.int32)
          for kk in range(D // _LANES):
            x = plsc.load_gather(slab_p, [rows_k[kk], col])
            outb_p[j, pl.ds(h * D + kk * _LANES, _LANES)] = x

    for i in range(8):
      fire_in(i)

    def loop_body(i, _):
      wait_in(i & 7)

      @pl.when(i + 8 < nslab)
      def _():
        fire_in(i + 8)

      @pl.when(i >= 4)
      def _():
        wait_out()

      transpose(i & 7, i & 3, 64)
      vg = wid + i * _NW
      pltpu.async_copy(outb_v.at[i & 3],
                       scr_hbm.at[pl.ds(vg * 64, 64), :], sem_out)
      return ()

    lax.fori_loop(0, nslab, loop_body, ())

    for _k in range(4):
      wait_out()

  return k


def _gather_dot_kernel(B, L, D):
  nb_per_w = B // _NW
  n_chunks = nb_per_w // _CHUNK
  rows_per_chunk = _CHUNK * L
  n_idx_rows = rows_per_chunk // 128
  nk = D // _LANES

  @functools.partial(
      pl.kernel,
      out_type=jax.ShapeDtypeStruct((B * L,), jnp.float32),
      mesh=plsc.VectorSubcoreMesh(**_MESH),
      compiler_params=_PARAMS,
      scratch_types=[
          pltpu.VMEM((_CHUNK + _LANES,), jnp.int32),
          pltpu.VMEM((_CHUNK,), jnp.int32),
          pltpu.VMEM((rows_per_chunk + _LANES,), jnp.int32),
          pltpu.VMEM((n_idx_rows, 128), jnp.int32),
          pltpu.VMEM((_CHUNK, 2 * D), jnp.float32),
          pltpu.VMEM((rows_per_chunk, 2 * D), jnp.float32),
          pltpu.VMEM((rows_per_chunk,), jnp.float32),
          pltpu.SemaphoreType.DMA,
          pltpu.SemaphoreType.DMA,
      ],
  )
  def k(center_hbm, con_hbm, scru_hbm, scrv_hbm, out_hbm,
        craw_v, cidx_v, uraw_v, uidx_v, vrows_v, urows_v, res_v,
        sem_u, sem_v):
    wid = lax.axis_index("s") * _NC + lax.axis_index("c")
    lane = lax.iota(jnp.int32, _LANES)
    lane15 = lane == (_LANES - 1)
    perms = [(lane ^ (1 << p)).reshape(_LANES, 1) for p in range(4)]
    gdims = lax.GatherDimensionNumbers(
        offset_dims=(), collapsed_slice_dims=(0,), start_index_map=(0,))

    def lane_sum(x):
      for p in perms:
        x = x + lax.gather(x, p, gdims, (1,),
                           mode=lax.GatherScatterMode.PROMISE_IN_BOUNDS)
      return x

    def chunk_body(c, _):
      b_base = wid * nb_per_w + c * _CHUNK
      pltpu.sync_copy(center_hbm.at[pl.ds(b_base, _CHUNK)],
                      craw_v.at[pl.ds(0, _CHUNK)])
      pltpu.sync_copy(con_hbm.at[pl.ds(b_base * L, rows_per_chunk)],
                      uraw_v.at[pl.ds(0, rows_per_chunk)])
      for t in range(_CHUNK // _LANES):
        s = pl.ds(t * _LANES, _LANES)
        cidx_v[s] = craw_v[s] >> 1
      for r in range(n_idx_rows):
        for t in range(128 // _LANES):
          uidx_v[r, pl.ds(t * _LANES, _LANES)] = (
              uraw_v[pl.ds(r * 128 + t * _LANES, _LANES)] >> 1)
      cp_v = pltpu.async_copy(scrv_hbm.at[cidx_v], vrows_v, sem_v)
      for j in range(n_idx_rows):
        pltpu.async_copy(scru_hbm.at[uidx_v.at[j]],
                         urows_v.at[pl.ds(j * 128, 128)], sem_u)
      cp_v.wait()
      pltpu.make_async_copy(
          scru_hbm.at[pl.ds(0, rows_per_chunk), :], urows_v, sem_u).wait()

      def dot_body(b, _):
        vbase = (craw_v[pl.ds(b, _LANES)][0] & 1) * D
        vr = [vrows_v[b, pl.ds(vbase + kk * _LANES, _LANES)] for kk in range(nk)]
        upar0 = (uraw_v[pl.ds(b * L, _LANES)] & 1) * D
        upar1 = (uraw_v[pl.ds(b * L + _LANES, _LANES)] & 1) * D
        for l in range(L):
          row = b * L + l
          ubase = upar0[l] if l < _LANES else upar1[l - _LANES]
          acc = urows_v[row, pl.ds(ubase, _LANES)] * vr[0]
          for kk in range(1, nk):
            acc += urows_v[row, pl.ds(ubase + kk * _LANES, _LANES)] * vr[kk]
          tot = lane_sum(acc)
          plsc.store_scatter(
              res_v, [jnp.full((_LANES,), row, jnp.int32)], tot, mask=lane15)
        return ()

      lax.fori_loop(0, _CHUNK, dot_body, ())
      pltpu.sync_copy(res_v, out_hbm.at[pl.ds(b_base * L, rows_per_chunk)])
      return ()

    lax.fori_loop(0, n_chunks, chunk_body, ())

  return k


def kernel(center, con_neg, emb_u, emb_v):
  B, L = con_neg.shape
  V, D = emb_u.shape
  assert B % (_NW * _CHUNK) == 0 and (_CHUNK * L) % 128 == 0
  assert D % _LANES == 0 and V % 2 == 0
  center_flat = center.reshape(B).astype(jnp.int32)
  con_flat = con_neg.reshape(B * L).astype(jnp.int32)
  relayout = _relayout_kernel(V, D)
  scr_u = relayout(emb_u.T)
  scr_v = relayout(emb_v.T)
  out = _gather_dot_kernel(B, L, D)(center_flat, con_flat, scr_u, scr_v)
  return out.reshape(B, 1, L)


# speedup vs baseline: 2.4581x; 2.4468x over previous
import functools

import jax
import jax.numpy as jnp
from jax import lax
from jax.experimental import pallas as pl
from jax.experimental.pallas import tpu as pltpu
from jax.experimental.pallas import tpu_sc as plsc

_NC = 2
_NS = 16
_NW = _NC * _NS
_LANES = 16

_CHUNK = 32

_PARAMS = pltpu.CompilerParams(use_tc_tiling_on_sc=False,
                               needs_layout_passes=False,
                               disable_bounds_checks=True)


def _sc_kernel(B, L, D):
  nb_per_w = B // _NW
  n_chunks = nb_per_w // _CHUNK
  rows_per_chunk = _CHUNK * L
  n_idx_rows = rows_per_chunk // 128
  nk = D // _LANES

  mesh = plsc.VectorSubcoreMesh(core_axis_name="c", subcore_axis_name="s")

  @functools.partial(
      pl.kernel,
      out_type=jax.ShapeDtypeStruct((B * L,), jnp.float32),
      mesh=mesh,
      compiler_params=_PARAMS,
      scratch_types=[
          pltpu.VMEM((_CHUNK,), jnp.int32),
          pltpu.VMEM((n_idx_rows, 128), jnp.int32),
          pltpu.VMEM((_CHUNK, D), jnp.float32),
          pltpu.VMEM((rows_per_chunk, D), jnp.float32),
          pltpu.VMEM((rows_per_chunk,), jnp.float32),
          pltpu.SemaphoreType.DMA,
          pltpu.SemaphoreType.DMA,
      ],
  )
  def k(center_hbm, con_hbm, emb_u_hbm, emb_v_hbm, out_hbm,
        cidx_v, uidx_v, vrows_v, urows_v, res_v, sem_u, sem_v):
    wid = lax.axis_index("s") * _NC + lax.axis_index("c")
    lane = lax.iota(jnp.int32, _LANES)
    lane15 = lane == (_LANES - 1)

    def chunk_body(c, _):
      b_base = wid * nb_per_w + c * _CHUNK
      pltpu.sync_copy(center_hbm.at[pl.ds(b_base, _CHUNK)], cidx_v)
      for j in range(n_idx_rows):
        pltpu.sync_copy(con_hbm.at[pl.ds(b_base * L + j * 128, 128)],
                        uidx_v.at[j])
      cp_v = pltpu.async_copy(emb_v_hbm.at[cidx_v], vrows_v, sem_v)
      for j in range(n_idx_rows):
        pltpu.async_copy(emb_u_hbm.at[uidx_v.at[j]],
                         urows_v.at[pl.ds(j * 128, 128)], sem_u)
      cp_v.wait()
      pltpu.make_async_copy(
          emb_u_hbm.at[pl.ds(0, rows_per_chunk), :], urows_v, sem_u).wait()

      def dot_body(b, _):
        vr = [vrows_v[b, pl.ds(kk * _LANES, _LANES)] for kk in range(nk)]
        for l in range(L):
          row = b * L + l
          acc = urows_v[row, pl.ds(0, _LANES)] * vr[0]
          for kk in range(1, nk):
            acc += urows_v[row, pl.ds(kk * _LANES, _LANES)] * vr[kk]
          tot = plsc.cumsum(acc)
          plsc.store_scatter(
              res_v, [jnp.full((_LANES,), row, jnp.int32)], tot, mask=lane15)
        return ()

      lax.fori_loop(0, _CHUNK, dot_body, ())
      pltpu.sync_copy(res_v, out_hbm.at[pl.ds(b_base * L, rows_per_chunk)])
      return ()

    lax.fori_loop(0, n_chunks, chunk_body, ())

  return k


def kernel(center, con_neg, emb_u, emb_v):
  B, L = con_neg.shape
  V, D = emb_u.shape
  assert B % (_NW * _CHUNK) == 0 and (_CHUNK * L) % 128 == 0
  assert D % _LANES == 0
  center_flat = center.reshape(B).astype(jnp.int32)
  con_flat = con_neg.reshape(B * L).astype(jnp.int32)
  out = _sc_kernel(B, L, D)(center_flat, con_flat, emb_u, emb_v)
  return out.reshape(B, 1, L)
